# baseline (speedup 1.0000x reference)
import functools
import jax
import jax.numpy as jnp
from jax.experimental import pallas as pl
from jax.experimental.pallas import tpu as pltpu

N = 10000
NP = 10240
E = 320000
D = 128
DE = 16
DK = 128
EB = 512
NGRID_E = E // EB
NB = 128
NGRID_N = NP // NB


def _gelu(v):
    return 0.5 * v * (1.0 + jax.lax.erf(v * 0.7071067811865476))


def _node_kernel(x_ref, lng_ref, lnb_ref, wtx_ref, wq_ref, bq_ref, wkt_ref,
                 bk_ref, a_ref, kq_ref, c_ref):
    x = x_ref[...]
    mean = jnp.mean(x, axis=-1, keepdims=True)
    var = jnp.mean(jnp.square(x - mean), axis=-1, keepdims=True)
    xn = (x - mean) / jnp.sqrt(var + 1e-5) * lng_ref[...] + lnb_ref[...]
    a_ref[...] = jnp.dot(xn, wtx_ref[...], preferred_element_type=jnp.float32)
    q = jnp.dot(xn, wq_ref[...], preferred_element_type=jnp.float32) + bq_ref[...]
    kq_ref[...] = jnp.dot(q, wkt_ref[...], preferred_element_type=jnp.float32)
    c_ref[...] = jnp.dot(q, bk_ref[...].T, preferred_element_type=jnp.float32).reshape(1, 1, NB)


def _node_precompute(xp, ln_g, ln_b, Wtx, Wq, bq, WkT, bk):
    grid = (NGRID_N,)
    out = pl.pallas_call(
        _node_kernel,
        grid=grid,
        in_specs=[
            pl.BlockSpec((NB, D), lambda i: (i, 0)),
            pl.BlockSpec((1, D), lambda i: (0, 0)),
            pl.BlockSpec((1, D), lambda i: (0, 0)),
            pl.BlockSpec((D, DK), lambda i: (0, 0)),
            pl.BlockSpec((D, DK), lambda i: (0, 0)),
            pl.BlockSpec((1, DK), lambda i: (0, 0)),
            pl.BlockSpec((DK, DK), lambda i: (0, 0)),
            pl.BlockSpec((1, DK), lambda i: (0, 0)),
        ],
        out_specs=[
            pl.BlockSpec((NB, DK), lambda i: (i, 0)),
            pl.BlockSpec((NB, DK), lambda i: (i, 0)),
            pl.BlockSpec((1, 1, NB), lambda i: (i, 0, 0)),
        ],
        out_shape=[
            jax.ShapeDtypeStruct((NP, DK), jnp.float32),
            jax.ShapeDtypeStruct((NP, DK), jnp.float32),
            jax.ShapeDtypeStruct((NGRID_N, 1, NB), jnp.float32),
        ],
    )(xp, ln_g.reshape(1, D), ln_b.reshape(1, D), Wtx, Wq, bq.reshape(1, DK),
      WkT, bk.reshape(1, DK))
    A, kq, c3 = out
    return A, kq, c3.reshape(NP)


def _edge_kernel(g_ref, kqg_ref, ev_ref, et_ref, cg_ref, wte_ref, bt_ref,
                 xt_ref, att_ref):
    r = EB // 128
    evp = jnp.dot(ev_ref[...], wte_ref[...], preferred_element_type=jnp.float32) + bt_ref[...]
    xt = _gelu(g_ref[...] + evp)
    xt_ref[...] = xt
    dot = jnp.sum(xt * kqg_ref[...], axis=-1).reshape(1, r, 128)
    att_ref[...] = ((dot + cg_ref[...]) * et_ref[...]) * (1.0 / jnp.sqrt(float(DK)))


def _edge_elementwise(G, KQg, ev, et3, cg3, Wte, bt):
    grid = (NGRID_E,)
    r = EB // 128
    out = pl.pallas_call(
        _edge_kernel,
        grid=grid,
        in_specs=[
            pl.BlockSpec((EB, DK), lambda i: (i, 0)),
            pl.BlockSpec((EB, DK), lambda i: (i, 0)),
            pl.BlockSpec((EB, DE), lambda i: (i, 0)),
            pl.BlockSpec((1, r, 128), lambda i: (i, 0, 0)),
            pl.BlockSpec((1, r, 128), lambda i: (i, 0, 0)),
            pl.BlockSpec((DE, DK), lambda i: (0, 0)),
            pl.BlockSpec((1, DK), lambda i: (0, 0)),
        ],
        out_specs=[
            pl.BlockSpec((EB, DK), lambda i: (i, 0)),
            pl.BlockSpec((1, r, 128), lambda i: (i, 0, 0)),
        ],
        out_shape=[
            jax.ShapeDtypeStruct((E, DK), jnp.float32),
            jax.ShapeDtypeStruct((NGRID_E, r, 128), jnp.float32),
        ],
    )(G, KQg, ev, et3, cg3, Wte, bt.reshape(1, DK))
    return out


def _final_kernel(s2_ref, srep_ref, x_ref, wv_ref, bv_ref, out_ref):
    s_sum = s2_ref[0] + s2_ref[1]
    w = srep_ref[...] / (srep_ref[...] + 1e-16)
    aggr = jnp.dot(s_sum, wv_ref[...], preferred_element_type=jnp.float32) + w * bv_ref[...]
    out_ref[...] = x_ref[...] + _gelu(aggr)


def _final(S2, s_rep, xp, Wv, bv):
    grid = (NGRID_N,)
    return pl.pallas_call(
        _final_kernel,
        grid=grid,
        in_specs=[
            pl.BlockSpec((2, NB, DK), lambda i: (0, i, 0)),
            pl.BlockSpec((NB, DK), lambda i: (i, 0)),
            pl.BlockSpec((NB, D), lambda i: (i, 0)),
            pl.BlockSpec((DK, DK), lambda i: (0, 0)),
            pl.BlockSpec((1, DK), lambda i: (0, 0)),
        ],
        out_specs=pl.BlockSpec((NB, D), lambda i: (i, 0)),
        out_shape=jax.ShapeDtypeStruct((NP, D), jnp.float32),
    )(S2, s_rep, xp, Wv, bv.reshape(1, DK))


def kernel(x, edge_index, edge_type, edge_vector, ln_g, ln_b, Wt, bt, Wk, bk,
           Wq, bq, Wv, bv):
    row = edge_index[0]
    col = edge_index[1]
    xp = jnp.pad(x, ((0, NP - N), (0, 0)))
    A, kq, c = _node_precompute(xp, ln_g, ln_b, Wt[:D], Wq, bq, Wk.T, bk)

    G = A[row]
    KQg = kq[col]
    r = EB // 128
    cg3 = c[col].reshape(NGRID_E, r, 128)

    et3 = edge_type.reshape(NGRID_E, r, 128)
    xT, att3 = _edge_elementwise(G, KQg, edge_vector, et3, cg3, Wt[D:], bt)
    att = att3.reshape(E)

    m = jax.ops.segment_max(att, col, num_segments=NP)
    out_e = jnp.exp(att - m[col])
    s = jax.ops.segment_sum(out_e, col, num_segments=NP)
    a_e = out_e / (s[col] + 1e-16)
    Sm = jax.ops.segment_sum(a_e[:, None] * xT, col, num_segments=NP)
    S2 = jnp.stack([Sm, jnp.zeros_like(Sm)])
    s_rep = jnp.broadcast_to(s[:, None], (NP, DK))

    out = _final(S2, s_rep, xp, Wv, bv)
    return out[:N]


# speedup vs baseline: 1.4420x; 1.4420x over previous
import functools
import jax
import jax.numpy as jnp
from jax import lax
from jax.experimental import pallas as pl
from jax.experimental.pallas import tpu as pltpu
from jax.experimental.pallas import tpu_sc as plsc

N = 10000
NP = 10240
E = 320000
D = 128
DE = 16
DK = 128
EB = 512
NGRID_E = E // EB
NB = 128
NGRID_N = NP // NB


def _gelu(v):
    return 0.5 * v * (1.0 + jax.lax.erf(v * 0.7071067811865476))


def _node_kernel(x_ref, lng_ref, lnb_ref, wtx_ref, wq_ref, bq_ref, wkt_ref,
                 bk_ref, a_ref, kq_ref, c_ref):
    x = x_ref[...]
    mean = jnp.mean(x, axis=-1, keepdims=True)
    var = jnp.mean(jnp.square(x - mean), axis=-1, keepdims=True)
    xn = (x - mean) / jnp.sqrt(var + 1e-5) * lng_ref[...] + lnb_ref[...]
    a_ref[...] = jnp.dot(xn, wtx_ref[...], preferred_element_type=jnp.float32)
    q = jnp.dot(xn, wq_ref[...], preferred_element_type=jnp.float32) + bq_ref[...]
    kq_ref[...] = jnp.dot(q, wkt_ref[...], preferred_element_type=jnp.float32)
    c_ref[...] = jnp.dot(q, bk_ref[...].T, preferred_element_type=jnp.float32).reshape(1, 1, NB)


def _node_precompute(xp, ln_g, ln_b, Wtx, Wq, bq, WkT, bk):
    grid = (NGRID_N,)
    out = pl.pallas_call(
        _node_kernel,
        grid=grid,
        in_specs=[
            pl.BlockSpec((NB, D), lambda i: (i, 0)),
            pl.BlockSpec((1, D), lambda i: (0, 0)),
            pl.BlockSpec((1, D), lambda i: (0, 0)),
            pl.BlockSpec((D, DK), lambda i: (0, 0)),
            pl.BlockSpec((D, DK), lambda i: (0, 0)),
            pl.BlockSpec((1, DK), lambda i: (0, 0)),
            pl.BlockSpec((DK, DK), lambda i: (0, 0)),
            pl.BlockSpec((1, DK), lambda i: (0, 0)),
        ],
        out_specs=[
            pl.BlockSpec((NB, DK), lambda i: (i, 0)),
            pl.BlockSpec((NB, DK), lambda i: (i, 0)),
            pl.BlockSpec((1, 1, NB), lambda i: (i, 0, 0)),
        ],
        out_shape=[
            jax.ShapeDtypeStruct((NP, DK), jnp.float32),
            jax.ShapeDtypeStruct((NP, DK), jnp.float32),
            jax.ShapeDtypeStruct((NGRID_N, 1, NB), jnp.float32),
        ],
    )(xp, ln_g.reshape(1, D), ln_b.reshape(1, D), Wtx, Wq, bq.reshape(1, DK),
      WkT, bk.reshape(1, DK))
    A, kq, c3 = out
    return A, kq, c3.reshape(NP)


SC_NC = 2
SC_NS = 16
SC_NW = SC_NC * SC_NS
EPT = E // SC_NW
GPB = 200
NGB = EPT // GPB


def _gather_body(a_hbm, kq_hbm, c_hbm, row_hbm, col_hbm,
                 g_out, kq_out, cg_out,
                 row_v, col_v, ctab_v, cg_v, g_buf, kq_buf, sem1, sem2):
    wid = lax.axis_index("s") * SC_NC + lax.axis_index("c")
    ebase = wid * EPT
    pltpu.sync_copy(row_hbm.at[pl.ds(ebase, EPT)], row_v)
    pltpu.sync_copy(col_hbm.at[pl.ds(ebase, EPT)], col_v)
    pltpu.sync_copy(c_hbm, ctab_v)

    def cg_step(i, _):
        idx = col_v[pl.ds(i * 16, 16)]
        cg_v[pl.ds(i * 16, 16)] = plsc.load_gather(ctab_v, [idx])
        return 0

    lax.fori_loop(0, EPT // 16, cg_step, 0)
    pltpu.sync_copy(cg_v, cg_out.at[pl.ds(ebase, EPT)])

    def batch_step(j, _):
        off = j * GPB
        pltpu.async_copy(a_hbm.at[row_v.at[pl.ds(off, GPB)]], g_buf, sem1)
        pltpu.async_copy(kq_hbm.at[col_v.at[pl.ds(off, GPB)]], kq_buf, sem2).wait()
        pltpu.make_async_copy(a_hbm.at[row_v.at[pl.ds(off, GPB)]], g_buf, sem1).wait()
        pltpu.sync_copy(g_buf, g_out.at[pl.ds(ebase + off, GPB)])
        pltpu.sync_copy(kq_buf, kq_out.at[pl.ds(ebase + off, GPB)])
        return 0

    lax.fori_loop(0, NGB, batch_step, 0)


def _sc_gather(A, kq, c, row, col):
    mesh = plsc.VectorSubcoreMesh(core_axis_name="c", subcore_axis_name="s",
                                  num_cores=SC_NC, num_subcores=SC_NS)
    f = pl.kernel(
        _gather_body,
        out_type=[
            jax.ShapeDtypeStruct((E, DK), jnp.float32),
            jax.ShapeDtypeStruct((E, DK), jnp.float32),
            jax.ShapeDtypeStruct((E,), jnp.float32),
        ],
        mesh=mesh,
        compiler_params=pltpu.CompilerParams(
            needs_layout_passes=False, use_tc_tiling_on_sc=False),
        scratch_types=[
            pltpu.VMEM((EPT,), jnp.int32),
            pltpu.VMEM((EPT,), jnp.int32),
            pltpu.VMEM((NP,), jnp.float32),
            pltpu.VMEM((EPT,), jnp.float32),
            pltpu.VMEM((GPB, DK), jnp.float32),
            pltpu.VMEM((GPB, DK), jnp.float32),
            pltpu.SemaphoreType.DMA,
            pltpu.SemaphoreType.DMA,
        ],
    )
    return f(A, kq, c, row, col)


def _edge_kernel(g_ref, kqg_ref, ev_ref, et_ref, cg_ref, wte_ref, bt_ref,
                 xt_ref, att_ref):
    r = EB // 128
    evp = jnp.dot(ev_ref[...], wte_ref[...], preferred_element_type=jnp.float32) + bt_ref[...]
    xt = _gelu(g_ref[...] + evp)
    xt_ref[...] = xt
    dot = jnp.sum(xt * kqg_ref[...], axis=-1).reshape(1, r, 128)
    att_ref[...] = ((dot + cg_ref[...]) * et_ref[...]) * (1.0 / jnp.sqrt(float(DK)))


def _edge_elementwise(G, KQg, ev, et3, cg3, Wte, bt):
    grid = (NGRID_E,)
    r = EB // 128
    out = pl.pallas_call(
        _edge_kernel,
        grid=grid,
        in_specs=[
            pl.BlockSpec((EB, DK), lambda i: (i, 0)),
            pl.BlockSpec((EB, DK), lambda i: (i, 0)),
            pl.BlockSpec((EB, DE), lambda i: (i, 0)),
            pl.BlockSpec((1, r, 128), lambda i: (i, 0, 0)),
            pl.BlockSpec((1, r, 128), lambda i: (i, 0, 0)),
            pl.BlockSpec((DE, DK), lambda i: (0, 0)),
            pl.BlockSpec((1, DK), lambda i: (0, 0)),
        ],
        out_specs=[
            pl.BlockSpec((EB, DK), lambda i: (i, 0)),
            pl.BlockSpec((1, r, 128), lambda i: (i, 0, 0)),
        ],
        out_shape=[
            jax.ShapeDtypeStruct((E, DK), jnp.float32),
            jax.ShapeDtypeStruct((NGRID_E, r, 128), jnp.float32),
        ],
    )(G, KQg, ev, et3, cg3, Wte, bt.reshape(1, DK))
    return out


def _final_kernel(s2_ref, srep_ref, x_ref, wv_ref, bv_ref, out_ref):
    s_sum = s2_ref[0] + s2_ref[1]
    w = srep_ref[...] / (srep_ref[...] + 1e-16)
    aggr = jnp.dot(s_sum, wv_ref[...], preferred_element_type=jnp.float32) + w * bv_ref[...]
    out_ref[...] = x_ref[...] + _gelu(aggr)


def _final(S2, s_rep, xp, Wv, bv):
    grid = (NGRID_N,)
    return pl.pallas_call(
        _final_kernel,
        grid=grid,
        in_specs=[
            pl.BlockSpec((2, NB, DK), lambda i: (0, i, 0)),
            pl.BlockSpec((NB, DK), lambda i: (i, 0)),
            pl.BlockSpec((NB, D), lambda i: (i, 0)),
            pl.BlockSpec((DK, DK), lambda i: (0, 0)),
            pl.BlockSpec((1, DK), lambda i: (0, 0)),
        ],
        out_specs=pl.BlockSpec((NB, D), lambda i: (i, 0)),
        out_shape=jax.ShapeDtypeStruct((NP, D), jnp.float32),
    )(S2, s_rep, xp, Wv, bv.reshape(1, DK))


def kernel(x, edge_index, edge_type, edge_vector, ln_g, ln_b, Wt, bt, Wk, bk,
           Wq, bq, Wv, bv):
    row = edge_index[0]
    col = edge_index[1]
    xp = jnp.pad(x, ((0, NP - N), (0, 0)))
    A, kq, c = _node_precompute(xp, ln_g, ln_b, Wt[:D], Wq, bq, Wk.T, bk)

    G, KQg, cg = _sc_gather(A, kq, c, row, col)
    r = EB // 128
    cg3 = cg.reshape(NGRID_E, r, 128)

    et3 = edge_type.reshape(NGRID_E, r, 128)
    xT, att3 = _edge_elementwise(G, KQg, edge_vector, et3, cg3, Wt[D:], bt)
    att = att3.reshape(E)

    m = jax.ops.segment_max(att, col, num_segments=NP)
    out_e = jnp.exp(att - m[col])
    s = jax.ops.segment_sum(out_e, col, num_segments=NP)
    a_e = out_e / (s[col] + 1e-16)
    Sm = jax.ops.segment_sum(a_e[:, None] * xT, col, num_segments=NP)
    S2 = jnp.stack([Sm, jnp.zeros_like(Sm)])
    s_rep = jnp.broadcast_to(s[:, None], (NP, DK))

    out = _final(S2, s_rep, xp, Wv, bv)
    return out[:N]


# speedup vs baseline: 3.0275x; 2.0995x over previous
import functools
import jax
import jax.numpy as jnp
from jax import lax
from jax.experimental import pallas as pl
from jax.experimental.pallas import tpu as pltpu
from jax.experimental.pallas import tpu_sc as plsc

N = 10000
NP = 10240
E = 320000
D = 128
DE = 16
DK = 128
EB = 512
NGRID_E = E // EB
NB = 128
NGRID_N = NP // NB


def _gelu(v):
    return 0.5 * v * (1.0 + jax.lax.erf(v * 0.7071067811865476))


def _node_kernel(x_ref, lng_ref, lnb_ref, wtx_ref, wq_ref, bq_ref, wkt_ref,
                 bk_ref, a_ref, kq_ref, c_ref):
    x = x_ref[...]
    mean = jnp.mean(x, axis=-1, keepdims=True)
    var = jnp.mean(jnp.square(x - mean), axis=-1, keepdims=True)
    xn = (x - mean) / jnp.sqrt(var + 1e-5) * lng_ref[...] + lnb_ref[...]
    a_ref[...] = jnp.dot(xn, wtx_ref[...], preferred_element_type=jnp.float32)
    q = jnp.dot(xn, wq_ref[...], preferred_element_type=jnp.float32) + bq_ref[...]
    kq_ref[...] = jnp.dot(q, wkt_ref[...], preferred_element_type=jnp.float32)
    c_ref[...] = jnp.dot(q, bk_ref[...].T, preferred_element_type=jnp.float32).reshape(1, 1, NB)


def _node_precompute(xp, ln_g, ln_b, Wtx, Wq, bq, WkT, bk):
    grid = (NGRID_N,)
    out = pl.pallas_call(
        _node_kernel,
        grid=grid,
        in_specs=[
            pl.BlockSpec((NB, D), lambda i: (i, 0)),
            pl.BlockSpec((1, D), lambda i: (0, 0)),
            pl.BlockSpec((1, D), lambda i: (0, 0)),
            pl.BlockSpec((D, DK), lambda i: (0, 0)),
            pl.BlockSpec((D, DK), lambda i: (0, 0)),
            pl.BlockSpec((1, DK), lambda i: (0, 0)),
            pl.BlockSpec((DK, DK), lambda i: (0, 0)),
            pl.BlockSpec((1, DK), lambda i: (0, 0)),
        ],
        out_specs=[
            pl.BlockSpec((NB, DK), lambda i: (i, 0)),
            pl.BlockSpec((NB, DK), lambda i: (i, 0)),
            pl.BlockSpec((1, 1, NB), lambda i: (i, 0, 0)),
        ],
        out_shape=[
            jax.ShapeDtypeStruct((NP, DK), jnp.float32),
            jax.ShapeDtypeStruct((NP, DK), jnp.float32),
            jax.ShapeDtypeStruct((NGRID_N, 1, NB), jnp.float32),
        ],
    )(xp, ln_g.reshape(1, D), ln_b.reshape(1, D), Wtx, Wq, bq.reshape(1, DK),
      WkT, bk.reshape(1, DK))
    A, kq, c3 = out
    return A, kq, c3.reshape(NP)


SC_NC = 2
SC_NS = 16
SC_NW = SC_NC * SC_NS
EPT = E // SC_NW
GPB = 200
NGB = EPT // GPB


def _gather_body(a_hbm, kq_hbm, c_hbm, row_hbm, col_hbm,
                 g_out, kq_out, cg_out,
                 row_v, col_v, ctab_v, cg_v, g_buf, kq_buf, sem1, sem2):
    wid = lax.axis_index("s") * SC_NC + lax.axis_index("c")
    ebase = wid * EPT
    pltpu.sync_copy(row_hbm.at[pl.ds(ebase, EPT)], row_v)
    pltpu.sync_copy(col_hbm.at[pl.ds(ebase, EPT)], col_v)
    pltpu.sync_copy(c_hbm, ctab_v)

    def cg_step(i, _):
        idx = col_v[pl.ds(i * 16, 16)]
        cg_v[pl.ds(i * 16, 16)] = plsc.load_gather(ctab_v, [idx])
        return 0

    lax.fori_loop(0, EPT // 16, cg_step, 0)
    pltpu.sync_copy(cg_v, cg_out.at[pl.ds(ebase, EPT)])

    def batch_step(j, _):
        off = j * GPB
        pltpu.async_copy(a_hbm.at[row_v.at[pl.ds(off, GPB)]], g_buf, sem1)
        pltpu.async_copy(kq_hbm.at[col_v.at[pl.ds(off, GPB)]], kq_buf, sem2).wait()
        pltpu.make_async_copy(a_hbm.at[row_v.at[pl.ds(off, GPB)]], g_buf, sem1).wait()
        pltpu.sync_copy(g_buf, g_out.at[pl.ds(ebase + off, GPB)])
        pltpu.sync_copy(kq_buf, kq_out.at[pl.ds(ebase + off, GPB)])
        return 0

    lax.fori_loop(0, NGB, batch_step, 0)


def _sc_gather(A, kq, c, row, col):
    mesh = plsc.VectorSubcoreMesh(core_axis_name="c", subcore_axis_name="s",
                                  num_cores=SC_NC, num_subcores=SC_NS)
    f = pl.kernel(
        _gather_body,
        out_type=[
            jax.ShapeDtypeStruct((E, DK), jnp.float32),
            jax.ShapeDtypeStruct((E, DK), jnp.float32),
            jax.ShapeDtypeStruct((E,), jnp.float32),
        ],
        mesh=mesh,
        compiler_params=pltpu.CompilerParams(
            needs_layout_passes=False, use_tc_tiling_on_sc=False),
        scratch_types=[
            pltpu.VMEM((EPT,), jnp.int32),
            pltpu.VMEM((EPT,), jnp.int32),
            pltpu.VMEM((NP,), jnp.float32),
            pltpu.VMEM((EPT,), jnp.float32),
            pltpu.VMEM((GPB, DK), jnp.float32),
            pltpu.VMEM((GPB, DK), jnp.float32),
            pltpu.SemaphoreType.DMA,
            pltpu.SemaphoreType.DMA,
        ],
    )
    return f(A, kq, c, row, col)


def _edge_kernel(g_ref, kqg_ref, ev_ref, et_ref, cg_ref, wte_ref, bt_ref,
                 xt_ref, att_ref):
    r = EB // 128
    evp = jnp.dot(ev_ref[...], wte_ref[...], preferred_element_type=jnp.float32) + bt_ref[...]
    xt = _gelu(g_ref[...] + evp)
    xt_ref[...] = xt
    dot = jnp.sum(xt * kqg_ref[...], axis=-1).reshape(1, r, 128)
    att_ref[...] = ((dot + cg_ref[...]) * et_ref[...]) * (1.0 / jnp.sqrt(float(DK)))


def _edge_elementwise(G, KQg, ev, et3, cg3, Wte, bt):
    grid = (NGRID_E,)
    r = EB // 128
    out = pl.pallas_call(
        _edge_kernel,
        grid=grid,
        in_specs=[
            pl.BlockSpec((EB, DK), lambda i: (i, 0)),
            pl.BlockSpec((EB, DK), lambda i: (i, 0)),
            pl.BlockSpec((EB, DE), lambda i: (i, 0)),
            pl.BlockSpec((1, r, 128), lambda i: (i, 0, 0)),
            pl.BlockSpec((1, r, 128), lambda i: (i, 0, 0)),
            pl.BlockSpec((DE, DK), lambda i: (0, 0)),
            pl.BlockSpec((1, DK), lambda i: (0, 0)),
        ],
        out_specs=[
            pl.BlockSpec((EB, DK), lambda i: (i, 0)),
            pl.BlockSpec((1, r, 128), lambda i: (i, 0, 0)),
        ],
        out_shape=[
            jax.ShapeDtypeStruct((E, DK), jnp.float32),
            jax.ShapeDtypeStruct((NGRID_E, r, 128), jnp.float32),
        ],
    )(G, KQg, ev, et3, cg3, Wte, bt.reshape(1, DK))
    return out


SAW = 144
SPB = 80
NSB = EPT // SPB
NPT = NP // SC_NS
NPH = NP // 2
NZR = NPH // SC_NS


def _softmax_body(att_hbm, colf_hbm, xt_hbm,
                  s_out, mu_out,
                  m_v, tmp_v, acc_v, mu8f,
                  att_b, col_b, idx_b, xt_buf, w_buf, msh, ssh):
    cid = lax.axis_index("c")
    sid = lax.axis_index("s")
    wid = sid * SC_NC + cid
    ebase = wid * EPT
    nbase = sid * NPT
    neg = jnp.float32(-1e30)
    zero16 = jnp.zeros((16,), jnp.float32)

    def fill(i, _):
        m_v[pl.ds(i * 16, 16)] = jnp.full((16,), neg)
        return 0
    lax.fori_loop(0, NP // 16, fill, 0)

    def maxbatch(j, _):
        pltpu.sync_copy(att_hbm.at[pl.ds(ebase + j * SPB, SPB)], att_b)
        pltpu.sync_copy(colf_hbm.at[pl.ds(ebase + j * SPB, SPB)], col_b)

        def rmw(i, _):
            sl = pl.ds(i * 16, 16)
            a16 = att_b[sl]
            c16 = col_b[sl]
            cur = plsc.load_gather(m_v, [c16])

            def cond(cur):
                return jnp.any(a16 > cur)

            def body(cur):
                plsc.store_scatter(m_v, [c16], a16, mask=a16 > cur)
                return plsc.load_gather(m_v, [c16])

            lax.while_loop(cond, body, cur)
            return 0
        lax.fori_loop(0, SPB // 16, rmw, 0)
        return 0
    lax.fori_loop(0, NSB, maxbatch, 0)

    pltpu.sync_copy(m_v, msh.at[sid])
    plsc.subcore_barrier()
    pltpu.sync_copy(msh.at[0, pl.ds(nbase, NPT)], acc_v)

    def comb(j, _):
        pltpu.sync_copy(msh.at[j, pl.ds(nbase, NPT)], tmp_v)

        def mx(k, _):
            sl = pl.ds(k * 16, 16)
            acc_v[sl] = jnp.maximum(acc_v[sl], tmp_v[sl])
            return 0
        lax.fori_loop(0, NPT // 16, mx, 0)
        return 0
    lax.fori_loop(1, SC_NS, comb, 0)

    def z8(k, _):
        mu8f[pl.ds(k * 16, 16)] = zero16
        return 0
    lax.fori_loop(0, NPT * 8 // 16, z8, 0)

    def sc8(k, _):
        v = acc_v[pl.ds(k * 16, 16)]
        idx = (lax.iota(jnp.int32, 16) + k * 16) * 8
        plsc.store_scatter(mu8f, [idx], v)
        return 0
    lax.fori_loop(0, NPT // 16, sc8, 0)
    pltpu.sync_copy(mu8f, mu_out.at[cid, pl.ds(nbase * 8, NPT * 8)])

    pltpu.sync_copy(acc_v, msh.at[SC_NS, pl.ds(nbase, NPT)])
    plsc.subcore_barrier()
    pltpu.sync_copy(msh.at[SC_NS], m_v)

    def phase(p, _):
        pbase = p * NPH

        def zw(k, _):
            e = k // (SAW // 16)
            q = k % (SAW // 16)
            w_buf[e, pl.ds(q * 16, 16)] = zero16
            return 0
        lax.fori_loop(0, SPB * (SAW // 16), zw, 0)

        def zs(k, _):
            pltpu.sync_copy(w_buf.at[pl.ds(0, 80)],
                            ssh.at[pl.ds(sid * NZR + k * 80, 80), :])
            return 0
        lax.fori_loop(0, NZR // 80, zs, 0)

        @pl.when(sid == 0)
        def _():
            pltpu.sync_copy(w_buf.at[pl.ds(0, 8)], ssh.at[pl.ds(NPH, 8), :])

        plsc.subcore_barrier()

        def batch(j, _):
            eoff = ebase + j * SPB
            pltpu.sync_copy(att_hbm.at[pl.ds(eoff, SPB)], att_b)
            pltpu.sync_copy(colf_hbm.at[pl.ds(eoff, SPB)], col_b)
            pltpu.sync_copy(xt_hbm.at[pl.ds(eoff, SPB)], xt_buf)

            def ex(i, _):
                sl = pl.ds(i * 16, 16)
                c16 = col_b[sl]
                mu16 = plsc.load_gather(m_v, [c16])
                att_b[sl] = jnp.exp(att_b[sl] - mu16)
                i16 = c16 - pbase
                ok = (i16 >= 0) & (i16 < NPH)
                idx_b[sl] = jnp.where(ok, i16, NPH)
                return 0
            lax.fori_loop(0, SPB // 16, ex, 0)

            def edge(e, _):
                ov = plsc.load_gather(
                    att_b, [jnp.full((16,), e, dtype=jnp.int32)])
                for d in range(DK // 16):
                    sl = pl.ds(d * 16, 16)
                    w_buf[e, sl] = xt_buf[e, sl] * ov
                w_buf[e, pl.ds(DK, 16)] = jnp.where(
                    lax.iota(jnp.int32, 16) == 0, ov, 0.0)
                return 0
            lax.fori_loop(0, SPB, edge, 0)
            pltpu.sync_copy(w_buf, ssh.at[idx_b], add=True)
            return 0
        lax.fori_loop(0, NSB, batch, 0)

        plsc.subcore_barrier()
        pltpu.sync_copy(ssh.at[pl.ds(sid * NZR, NZR), :],
                        s_out.at[cid, pl.ds(pbase + sid * NZR, NZR), :])
        plsc.subcore_barrier()
        return 0
    lax.fori_loop(0, 2, phase, 0)


def _sc_softmax_scatter(att, colf, xT):
    mesh = plsc.VectorSubcoreMesh(core_axis_name="c", subcore_axis_name="s",
                                  num_cores=SC_NC, num_subcores=SC_NS)
    f = pl.kernel(
        _softmax_body,
        out_type=[
            jax.ShapeDtypeStruct((SC_NC, NP, SAW), jnp.float32),
            jax.ShapeDtypeStruct((SC_NC, NP * 8), jnp.float32),
        ],
        mesh=mesh,
        compiler_params=pltpu.CompilerParams(
            needs_layout_passes=False, use_tc_tiling_on_sc=False),
        scratch_types=[
            pltpu.VMEM((NP,), jnp.float32),
            pltpu.VMEM((NPT,), jnp.float32),
            pltpu.VMEM((NPT,), jnp.float32),
            pltpu.VMEM((NPT * 8,), jnp.float32),
            pltpu.VMEM((SPB,), jnp.float32),
            pltpu.VMEM((SPB,), jnp.int32),
            pltpu.VMEM((SPB,), jnp.int32),
            pltpu.VMEM((SPB, DK), jnp.float32),
            pltpu.VMEM((SPB, SAW), jnp.float32),
            pltpu.VMEM_SHARED((SC_NS + 1, NP), jnp.float32),
            pltpu.VMEM_SHARED((NPH + 8, SAW), jnp.float32),
        ],
    )
    return f(att, colf, xT)


def _final_kernel(s2_ref, mu_ref, x_ref, wv_ref, bv_ref, out_ref):
    mu0 = mu_ref[0, :, 0:1]
    mu1 = mu_ref[1, :, 0:1]
    mu = jnp.maximum(mu0, mu1)
    r0 = jnp.exp(mu0 - mu)
    r1 = jnp.exp(mu1 - mu)
    s = r0 * s2_ref[0, :, DK:DK + 1] + r1 * s2_ref[1, :, DK:DK + 1]
    inv = 1.0 / (s + 1e-16)
    P = (r0 * inv) * s2_ref[0, :, 0:DK] + (r1 * inv) * s2_ref[1, :, 0:DK]
    aggr = jnp.dot(P, wv_ref[...], preferred_element_type=jnp.float32) \
        + (s * inv) * bv_ref[...]
    out_ref[...] = x_ref[...] + _gelu(aggr)


def _final(S2aug, mu8, xp, Wv, bv):
    grid = (NGRID_N,)
    return pl.pallas_call(
        _final_kernel,
        grid=grid,
        in_specs=[
            pl.BlockSpec((2, NB, SAW), lambda i: (0, i, 0)),
            pl.BlockSpec((2, NB, 8), lambda i: (0, i, 0)),
            pl.BlockSpec((NB, D), lambda i: (i, 0)),
            pl.BlockSpec((DK, DK), lambda i: (0, 0)),
            pl.BlockSpec((1, DK), lambda i: (0, 0)),
        ],
        out_specs=pl.BlockSpec((NB, D), lambda i: (i, 0)),
        out_shape=jax.ShapeDtypeStruct((NP, D), jnp.float32),
    )(S2aug, mu8, xp, Wv, bv.reshape(1, DK))


def kernel(x, edge_index, edge_type, edge_vector, ln_g, ln_b, Wt, bt, Wk, bk,
           Wq, bq, Wv, bv):
    row = edge_index[0]
    col = edge_index[1]
    xp = jnp.pad(x, ((0, NP - N), (0, 0)))
    A, kq, c = _node_precompute(xp, ln_g, ln_b, Wt[:D], Wq, bq, Wk.T, bk)

    G, KQg, cg = _sc_gather(A, kq, c, row, col)
    r = EB // 128
    cg3 = cg.reshape(NGRID_E, r, 128)

    et3 = edge_type.reshape(NGRID_E, r, 128)
    xT, att3 = _edge_elementwise(G, KQg, edge_vector, et3, cg3, Wt[D:], bt)
    att = att3.reshape(E)

    S2aug, mu8f = _sc_softmax_scatter(att, col, xT)

    out = _final(S2aug, mu8f.reshape(SC_NC, NP, 8), xp, Wv, bv)
    return out[:N]


# speedup vs baseline: 3.6825x; 1.2163x over previous
import functools
import jax
import jax.numpy as jnp
from jax import lax
from jax.experimental import pallas as pl
from jax.experimental.pallas import tpu as pltpu
from jax.experimental.pallas import tpu_sc as plsc

N = 10000
NP = 10240
E = 320000
D = 128
DE = 16
DK = 128
EB = 512
NGRID_E = E // EB
NB = 128
NGRID_N = NP // NB


def _gelu(v):
    return 0.5 * v * (1.0 + jax.lax.erf(v * 0.7071067811865476))


def _node_kernel(x_ref, lng_ref, lnb_ref, wtx_ref, wq_ref, bq_ref, wkt_ref,
                 bk_ref, a_ref, kq_ref, c_ref):
    x = x_ref[...]
    mean = jnp.mean(x, axis=-1, keepdims=True)
    var = jnp.mean(jnp.square(x - mean), axis=-1, keepdims=True)
    xn = (x - mean) / jnp.sqrt(var + 1e-5) * lng_ref[...] + lnb_ref[...]
    a_ref[...] = jnp.dot(xn, wtx_ref[...], preferred_element_type=jnp.float32)
    q = jnp.dot(xn, wq_ref[...], preferred_element_type=jnp.float32) + bq_ref[...]
    kq_ref[...] = jnp.dot(q, wkt_ref[...], preferred_element_type=jnp.float32)
    c_ref[...] = jnp.dot(q, bk_ref[...].T, preferred_element_type=jnp.float32).reshape(1, 1, NB)


def _node_precompute(xp, ln_g, ln_b, Wtx, Wq, bq, WkT, bk):
    grid = (NGRID_N,)
    out = pl.pallas_call(
        _node_kernel,
        grid=grid,
        in_specs=[
            pl.BlockSpec((NB, D), lambda i: (i, 0)),
            pl.BlockSpec((1, D), lambda i: (0, 0)),
            pl.BlockSpec((1, D), lambda i: (0, 0)),
            pl.BlockSpec((D, DK), lambda i: (0, 0)),
            pl.BlockSpec((D, DK), lambda i: (0, 0)),
            pl.BlockSpec((1, DK), lambda i: (0, 0)),
            pl.BlockSpec((DK, DK), lambda i: (0, 0)),
            pl.BlockSpec((1, DK), lambda i: (0, 0)),
        ],
        out_specs=[
            pl.BlockSpec((NB, DK), lambda i: (i, 0)),
            pl.BlockSpec((NB, DK), lambda i: (i, 0)),
            pl.BlockSpec((1, 1, NB), lambda i: (i, 0, 0)),
        ],
        out_shape=[
            jax.ShapeDtypeStruct((NP, DK), jnp.float32),
            jax.ShapeDtypeStruct((NP, DK), jnp.float32),
            jax.ShapeDtypeStruct((NGRID_N, 1, NB), jnp.float32),
        ],
    )(xp, ln_g.reshape(1, D), ln_b.reshape(1, D), Wtx, Wq, bq.reshape(1, DK),
      WkT, bk.reshape(1, DK))
    A, kq, c3 = out
    return A, kq, c3.reshape(NP)


SC_NC = 2
SC_NS = 16
SC_NW = SC_NC * SC_NS
EPT = E // SC_NW
GPB = 200
NGB = EPT // GPB


def _gather_body(a_hbm, kq_hbm, c_hbm, row_hbm, col_hbm,
                 g_out, kq_out, cg_out,
                 row_v, col_v, ctab_v, cg_v, g_buf, kq_buf, sem1, sem2):
    wid = lax.axis_index("s") * SC_NC + lax.axis_index("c")
    ebase = wid * EPT
    pltpu.sync_copy(row_hbm.at[pl.ds(ebase, EPT)], row_v)
    pltpu.sync_copy(col_hbm.at[pl.ds(ebase, EPT)], col_v)
    pltpu.sync_copy(c_hbm, ctab_v)

    def cg_step(i, _):
        idx = col_v[pl.ds(i * 16, 16)]
        cg_v[pl.ds(i * 16, 16)] = plsc.load_gather(ctab_v, [idx])
        return 0

    lax.fori_loop(0, EPT // 16, cg_step, 0)
    pltpu.sync_copy(cg_v, cg_out.at[pl.ds(ebase, EPT)])

    def batch_step(j, _):
        off = j * GPB
        pltpu.async_copy(a_hbm.at[row_v.at[pl.ds(off, GPB)]], g_buf, sem1)
        pltpu.async_copy(kq_hbm.at[col_v.at[pl.ds(off, GPB)]], kq_buf, sem2).wait()
        pltpu.make_async_copy(a_hbm.at[row_v.at[pl.ds(off, GPB)]], g_buf, sem1).wait()
        pltpu.sync_copy(g_buf, g_out.at[pl.ds(ebase + off, GPB)])
        pltpu.sync_copy(kq_buf, kq_out.at[pl.ds(ebase + off, GPB)])
        return 0

    lax.fori_loop(0, NGB, batch_step, 0)


def _sc_gather(A, kq, c, row, col):
    mesh = plsc.VectorSubcoreMesh(core_axis_name="c", subcore_axis_name="s",
                                  num_cores=SC_NC, num_subcores=SC_NS)
    f = pl.kernel(
        _gather_body,
        out_type=[
            jax.ShapeDtypeStruct((E, DK), jnp.float32),
            jax.ShapeDtypeStruct((E, DK), jnp.float32),
            jax.ShapeDtypeStruct((E,), jnp.float32),
        ],
        mesh=mesh,
        compiler_params=pltpu.CompilerParams(
            needs_layout_passes=False, use_tc_tiling_on_sc=False),
        scratch_types=[
            pltpu.VMEM((EPT,), jnp.int32),
            pltpu.VMEM((EPT,), jnp.int32),
            pltpu.VMEM((NP,), jnp.float32),
            pltpu.VMEM((EPT,), jnp.float32),
            pltpu.VMEM((GPB, DK), jnp.float32),
            pltpu.VMEM((GPB, DK), jnp.float32),
            pltpu.SemaphoreType.DMA,
            pltpu.SemaphoreType.DMA,
        ],
    )
    return f(A, kq, c, row, col)


def _edge_kernel(g_ref, kqg_ref, ev_ref, et_ref, cg_ref, wte_ref, bt_ref,
                 xt_ref, att_ref):
    r = EB // 128
    evp = jnp.dot(ev_ref[...], wte_ref[...], preferred_element_type=jnp.float32) + bt_ref[...]
    xt = _gelu(g_ref[...] + evp)
    xt_ref[...] = xt
    dot = jnp.sum(xt * kqg_ref[...], axis=-1).reshape(1, r, 128)
    att_ref[...] = ((dot + cg_ref[...]) * et_ref[...]) * (1.0 / jnp.sqrt(float(DK)))


def _edge_elementwise(G, KQg, ev, et3, cg3, Wte, bt):
    grid = (NGRID_E,)
    r = EB // 128
    out = pl.pallas_call(
        _edge_kernel,
        grid=grid,
        in_specs=[
            pl.BlockSpec((EB, DK), lambda i: (i, 0)),
            pl.BlockSpec((EB, DK), lambda i: (i, 0)),
            pl.BlockSpec((EB, DE), lambda i: (i, 0)),
            pl.BlockSpec((1, r, 128), lambda i: (i, 0, 0)),
            pl.BlockSpec((1, r, 128), lambda i: (i, 0, 0)),
            pl.BlockSpec((DE, DK), lambda i: (0, 0)),
            pl.BlockSpec((1, DK), lambda i: (0, 0)),
        ],
        out_specs=[
            pl.BlockSpec((EB, DK), lambda i: (i, 0)),
            pl.BlockSpec((1, r, 128), lambda i: (i, 0, 0)),
        ],
        out_shape=[
            jax.ShapeDtypeStruct((E, DK), jnp.float32),
            jax.ShapeDtypeStruct((NGRID_E, r, 128), jnp.float32),
        ],
    )(G, KQg, ev, et3, cg3, Wte, bt.reshape(1, DK))
    return out


SW = 80
DH = 64
SPB = 80
NSB = EPT // SPB
NPT = NP // SC_NS


def _softmax_body(att_hbm, colf_hbm, xt_hbm,
                  s1_out, s2_out, mu_out,
                  m_v, tmp_v, acc_v, mu8f,
                  att_b, col_b, xt_buf, w_buf, msh, ssh):
    cid = lax.axis_index("c")
    sid = lax.axis_index("s")
    wid = sid * SC_NC + cid
    ebase = wid * EPT
    nbase = sid * NPT
    neg = jnp.float32(-1e30)
    zero16 = jnp.zeros((16,), jnp.float32)

    def fill(i, _):
        m_v[pl.ds(i * 16, 16)] = jnp.full((16,), neg)
        return 0
    lax.fori_loop(0, NP // 16, fill, 0)

    def maxbatch(j, _):
        pltpu.sync_copy(att_hbm.at[pl.ds(ebase + j * SPB, SPB)], att_b)
        pltpu.sync_copy(colf_hbm.at[pl.ds(ebase + j * SPB, SPB)], col_b)

        def rmw(i, _):
            sl = pl.ds(i * 16, 16)
            a16 = att_b[sl]
            c16 = col_b[sl]
            cur = plsc.load_gather(m_v, [c16])

            def cond(cur):
                return jnp.any(a16 > cur)

            def body(cur):
                plsc.store_scatter(m_v, [c16], a16, mask=a16 > cur)
                return plsc.load_gather(m_v, [c16])

            lax.while_loop(cond, body, cur)
            return 0
        lax.fori_loop(0, SPB // 16, rmw, 0)
        return 0
    lax.fori_loop(0, NSB, maxbatch, 0)

    pltpu.sync_copy(m_v, msh.at[sid])
    plsc.subcore_barrier()
    pltpu.sync_copy(msh.at[0, pl.ds(nbase, NPT)], acc_v)

    def comb(j, _):
        pltpu.sync_copy(msh.at[j, pl.ds(nbase, NPT)], tmp_v)

        def mx(k, _):
            sl = pl.ds(k * 16, 16)
            acc_v[sl] = jnp.maximum(acc_v[sl], tmp_v[sl])
            return 0
        lax.fori_loop(0, NPT // 16, mx, 0)
        return 0
    lax.fori_loop(1, SC_NS, comb, 0)

    def z8(k, _):
        mu8f[pl.ds(k * 16, 16)] = zero16
        return 0
    lax.fori_loop(0, NPT * 8 // 16, z8, 0)

    def sc8(k, _):
        v = acc_v[pl.ds(k * 16, 16)]
        idx = (lax.iota(jnp.int32, 16) + k * 16) * 8
        plsc.store_scatter(mu8f, [idx], v)
        return 0
    lax.fori_loop(0, NPT // 16, sc8, 0)
    pltpu.sync_copy(mu8f, mu_out.at[cid, pl.ds(nbase * 8, NPT * 8)])

    pltpu.sync_copy(acc_v, msh.at[SC_NS, pl.ds(nbase, NPT)])
    plsc.subcore_barrier()
    pltpu.sync_copy(msh.at[SC_NS], m_v)

    for p in range(2):
        out_hbm = s1_out if p == 0 else s2_out

        def zw(k, _):
            e = k // (SW // 16)
            q = k % (SW // 16)
            w_buf[e, pl.ds(q * 16, 16)] = zero16
            return 0
        lax.fori_loop(0, SPB * (SW // 16), zw, 0)

        def zs(k, _):
            pltpu.sync_copy(w_buf.at[pl.ds(0, 80)],
                            ssh.at[pl.ds(nbase + k * 80, 80), :])
            return 0
        lax.fori_loop(0, NPT // 80, zs, 0)

        plsc.subcore_barrier()

        def batch(j, _):
            eoff = ebase + j * SPB
            pltpu.sync_copy(att_hbm.at[pl.ds(eoff, SPB)], att_b)
            pltpu.sync_copy(colf_hbm.at[pl.ds(eoff, SPB)], col_b)
            pltpu.sync_copy(
                xt_hbm.at[pl.ds(eoff, SPB), pl.ds(p * DH, DH)], xt_buf)

            def ex(i, _):
                sl = pl.ds(i * 16, 16)
                mu16 = plsc.load_gather(m_v, [col_b[sl]])
                att_b[sl] = jnp.exp(att_b[sl] - mu16)
                return 0
            lax.fori_loop(0, SPB // 16, ex, 0)

            if p == 0:
                def edge(e, _):
                    ov = plsc.load_gather(
                        att_b, [jnp.full((16,), e, dtype=jnp.int32)])
                    for d in range(DH // 16):
                        sl = pl.ds(d * 16, 16)
                        w_buf[e, sl] = xt_buf[e, sl] * ov
                    w_buf[e, pl.ds(DH, 16)] = jnp.where(
                        lax.iota(jnp.int32, 16) == 0, ov, 0.0)
                    return 0
            else:
                def edge(e, _):
                    ov = plsc.load_gather(
                        att_b, [jnp.full((16,), e, dtype=jnp.int32)])
                    for d in range(DH // 16):
                        sl = pl.ds(d * 16, 16)
                        w_buf[e, sl] = xt_buf[e, sl] * ov
                    return 0
            lax.fori_loop(0, SPB, edge, 0)
            pltpu.sync_copy(w_buf, ssh.at[col_b], add=True)
            return 0
        lax.fori_loop(0, NSB, batch, 0)

        plsc.subcore_barrier()
        pltpu.sync_copy(ssh.at[pl.ds(nbase, NPT), :],
                        out_hbm.at[cid, pl.ds(nbase, NPT), :])
        plsc.subcore_barrier()


def _sc_softmax_scatter(att, colf, xT):
    mesh = plsc.VectorSubcoreMesh(core_axis_name="c", subcore_axis_name="s",
                                  num_cores=SC_NC, num_subcores=SC_NS)
    f = pl.kernel(
        _softmax_body,
        out_type=[
            jax.ShapeDtypeStruct((SC_NC, NP, SW), jnp.float32),
            jax.ShapeDtypeStruct((SC_NC, NP, SW), jnp.float32),
            jax.ShapeDtypeStruct((SC_NC, NP * 8), jnp.float32),
        ],
        mesh=mesh,
        compiler_params=pltpu.CompilerParams(
            needs_layout_passes=False, use_tc_tiling_on_sc=False),
        scratch_types=[
            pltpu.VMEM((NP,), jnp.float32),
            pltpu.VMEM((NPT,), jnp.float32),
            pltpu.VMEM((NPT,), jnp.float32),
            pltpu.VMEM((NPT * 8,), jnp.float32),
            pltpu.VMEM((SPB,), jnp.float32),
            pltpu.VMEM((SPB,), jnp.int32),
            pltpu.VMEM((SPB, DH), jnp.float32),
            pltpu.VMEM((SPB, SW), jnp.float32),
            pltpu.VMEM_SHARED((SC_NS + 1, NP), jnp.float32),
            pltpu.VMEM_SHARED((NP, SW), jnp.float32),
        ],
    )
    return f(att, colf, xT)


def _final_kernel(s1_ref, s2_ref, mu_ref, x_ref, wv_ref, bv_ref, out_ref):
    mu0 = mu_ref[0, :, 0:1]
    mu1 = mu_ref[1, :, 0:1]
    mu = jnp.maximum(mu0, mu1)
    r0 = jnp.exp(mu0 - mu)
    r1 = jnp.exp(mu1 - mu)
    s = r0 * s1_ref[0, :, DH:DH + 1] + r1 * s1_ref[1, :, DH:DH + 1]
    inv = 1.0 / (s + 1e-16)
    P1 = (r0 * inv) * s1_ref[0, :, 0:DH] + (r1 * inv) * s1_ref[1, :, 0:DH]
    P2 = (r0 * inv) * s2_ref[0, :, 0:DH] + (r1 * inv) * s2_ref[1, :, 0:DH]
    aggr = jnp.dot(P1, wv_ref[0:DH, :], preferred_element_type=jnp.float32) \
        + jnp.dot(P2, wv_ref[DH:DK, :], preferred_element_type=jnp.float32) \
        + (s * inv) * bv_ref[...]
    out_ref[...] = x_ref[...] + _gelu(aggr)


def _final(S1, S2, mu8, xp, Wv, bv):
    grid = (NGRID_N,)
    return pl.pallas_call(
        _final_kernel,
        grid=grid,
        in_specs=[
            pl.BlockSpec((2, NB, SW), lambda i: (0, i, 0)),
            pl.BlockSpec((2, NB, SW), lambda i: (0, i, 0)),
            pl.BlockSpec((2, NB, 8), lambda i: (0, i, 0)),
            pl.BlockSpec((NB, D), lambda i: (i, 0)),
            pl.BlockSpec((DK, DK), lambda i: (0, 0)),
            pl.BlockSpec((1, DK), lambda i: (0, 0)),
        ],
        out_specs=pl.BlockSpec((NB, D), lambda i: (i, 0)),
        out_shape=jax.ShapeDtypeStruct((NP, D), jnp.float32),
    )(S1, S2, mu8, xp, Wv, bv.reshape(1, DK))


def kernel(x, edge_index, edge_type, edge_vector, ln_g, ln_b, Wt, bt, Wk, bk,
           Wq, bq, Wv, bv):
    row = edge_index[0]
    col = edge_index[1]
    xp = jnp.pad(x, ((0, NP - N), (0, 0)))
    A, kq, c = _node_precompute(xp, ln_g, ln_b, Wt[:D], Wq, bq, Wk.T, bk)

    G, KQg, cg = _sc_gather(A, kq, c, row, col)
    r = EB // 128
    cg3 = cg.reshape(NGRID_E, r, 128)

    et3 = edge_type.reshape(NGRID_E, r, 128)
    xT, att3 = _edge_elementwise(G, KQg, edge_vector, et3, cg3, Wt[D:], bt)
    att = att3.reshape(E)

    S1, S2, mu8f = _sc_softmax_scatter(att, col, xT)

    out = _final(S1, S2, mu8f.reshape(SC_NC, NP, 8), xp, Wv, bv)
    return out[:N]


# speedup vs baseline: 3.7465x; 1.0174x over previous
import functools
import jax
import jax.numpy as jnp
from jax import lax
from jax.experimental import pallas as pl
from jax.experimental.pallas import tpu as pltpu
from jax.experimental.pallas import tpu_sc as plsc

N = 10000
NP = 10240
E = 320000
D = 128
DE = 16
DK = 128
EB = 512
NGRID_E = E // EB
NB = 128
NGRID_N = NP // NB


def _gelu(v):
    return 0.5 * v * (1.0 + jax.lax.erf(v * 0.7071067811865476))


def _node_kernel(x_ref, lng_ref, lnb_ref, wtx_ref, wq_ref, bq_ref, wkt_ref,
                 bk_ref, a_ref, kq_ref, c_ref):
    x = x_ref[...]
    mean = jnp.mean(x, axis=-1, keepdims=True)
    var = jnp.mean(jnp.square(x - mean), axis=-1, keepdims=True)
    xn = (x - mean) / jnp.sqrt(var + 1e-5) * lng_ref[...] + lnb_ref[...]
    a_ref[...] = jnp.dot(xn, wtx_ref[...], preferred_element_type=jnp.float32)
    q = jnp.dot(xn, wq_ref[...], preferred_element_type=jnp.float32) + bq_ref[...]
    kq_ref[...] = jnp.dot(q, wkt_ref[...], preferred_element_type=jnp.float32)
    c_ref[...] = jnp.dot(q, bk_ref[...].T, preferred_element_type=jnp.float32).reshape(1, 1, NB)


def _node_precompute(xp, ln_g, ln_b, Wtx, Wq, bq, WkT, bk):
    grid = (NGRID_N,)
    out = pl.pallas_call(
        _node_kernel,
        grid=grid,
        in_specs=[
            pl.BlockSpec((NB, D), lambda i: (i, 0)),
            pl.BlockSpec((1, D), lambda i: (0, 0)),
            pl.BlockSpec((1, D), lambda i: (0, 0)),
            pl.BlockSpec((D, DK), lambda i: (0, 0)),
            pl.BlockSpec((D, DK), lambda i: (0, 0)),
            pl.BlockSpec((1, DK), lambda i: (0, 0)),
            pl.BlockSpec((DK, DK), lambda i: (0, 0)),
            pl.BlockSpec((1, DK), lambda i: (0, 0)),
        ],
        out_specs=[
            pl.BlockSpec((NB, DK), lambda i: (i, 0)),
            pl.BlockSpec((NB, DK), lambda i: (i, 0)),
            pl.BlockSpec((1, 1, NB), lambda i: (i, 0, 0)),
        ],
        out_shape=[
            jax.ShapeDtypeStruct((NP, DK), jnp.float32),
            jax.ShapeDtypeStruct((NP, DK), jnp.float32),
            jax.ShapeDtypeStruct((NGRID_N, 1, NB), jnp.float32),
        ],
    )(xp, ln_g.reshape(1, D), ln_b.reshape(1, D), Wtx, Wq, bq.reshape(1, DK),
      WkT, bk.reshape(1, DK))
    A, kq, c3 = out
    return A, kq, c3.reshape(NP)


SC_NC = 2
SC_NS = 16
SC_NW = SC_NC * SC_NS
EPT = E // SC_NW
GPB = 200
NGB = EPT // GPB


def _gather_body(a_hbm, kq_hbm, c_hbm, row_hbm, col_hbm,
                 g_out, kq_out, cg_out,
                 row_v, col_v, ctab_v, cg_v, g_buf, kq_buf, sem1, sem2):
    wid = lax.axis_index("s") * SC_NC + lax.axis_index("c")
    ebase = wid * EPT
    pltpu.sync_copy(row_hbm.at[pl.ds(ebase, EPT)], row_v)
    pltpu.sync_copy(col_hbm.at[pl.ds(ebase, EPT)], col_v)
    pltpu.sync_copy(c_hbm, ctab_v)

    def cg_step(i, _):
        idx = col_v[pl.ds(i * 16, 16)]
        cg_v[pl.ds(i * 16, 16)] = plsc.load_gather(ctab_v, [idx])
        return 0

    lax.fori_loop(0, EPT // 16, cg_step, 0)
    pltpu.sync_copy(cg_v, cg_out.at[pl.ds(ebase, EPT)])

    def batch_step(j, _):
        off = j * GPB
        pltpu.async_copy(a_hbm.at[row_v.at[pl.ds(off, GPB)]], g_buf, sem1)
        pltpu.async_copy(kq_hbm.at[col_v.at[pl.ds(off, GPB)]], kq_buf, sem2).wait()
        pltpu.make_async_copy(a_hbm.at[row_v.at[pl.ds(off, GPB)]], g_buf, sem1).wait()
        pltpu.sync_copy(g_buf, g_out.at[pl.ds(ebase + off, GPB)])
        pltpu.sync_copy(kq_buf, kq_out.at[pl.ds(ebase + off, GPB)])
        return 0

    lax.fori_loop(0, NGB, batch_step, 0)


def _sc_gather(A, kq, c, row, col):
    mesh = plsc.VectorSubcoreMesh(core_axis_name="c", subcore_axis_name="s",
                                  num_cores=SC_NC, num_subcores=SC_NS)
    f = pl.kernel(
        _gather_body,
        out_type=[
            jax.ShapeDtypeStruct((E, DK), jnp.float32),
            jax.ShapeDtypeStruct((E, DK), jnp.float32),
            jax.ShapeDtypeStruct((E,), jnp.float32),
        ],
        mesh=mesh,
        compiler_params=pltpu.CompilerParams(
            needs_layout_passes=False, use_tc_tiling_on_sc=False),
        scratch_types=[
            pltpu.VMEM((EPT,), jnp.int32),
            pltpu.VMEM((EPT,), jnp.int32),
            pltpu.VMEM((NP,), jnp.float32),
            pltpu.VMEM((EPT,), jnp.float32),
            pltpu.VMEM((GPB, DK), jnp.float32),
            pltpu.VMEM((GPB, DK), jnp.float32),
            pltpu.SemaphoreType.DMA,
            pltpu.SemaphoreType.DMA,
        ],
    )
    return f(A, kq, c, row, col)


def _edge_kernel(g_ref, kqg_ref, ev_ref, et_ref, cg_ref, wte_ref, bt_ref,
                 xt_ref, att_ref):
    r = EB // 128
    evp = jnp.dot(ev_ref[...], wte_ref[...], preferred_element_type=jnp.float32) + bt_ref[...]
    xt = _gelu(g_ref[...] + evp)
    xt_ref[...] = xt
    dot = jnp.sum(xt * kqg_ref[...], axis=-1).reshape(1, r, 128)
    att_ref[...] = ((dot + cg_ref[...]) * et_ref[...]) * (1.0 / jnp.sqrt(float(DK)))


def _edge_elementwise(G, KQg, ev, et3, cg3, Wte, bt):
    grid = (NGRID_E,)
    r = EB // 128
    out = pl.pallas_call(
        _edge_kernel,
        grid=grid,
        in_specs=[
            pl.BlockSpec((EB, DK), lambda i: (i, 0)),
            pl.BlockSpec((EB, DK), lambda i: (i, 0)),
            pl.BlockSpec((EB, DE), lambda i: (i, 0)),
            pl.BlockSpec((1, r, 128), lambda i: (i, 0, 0)),
            pl.BlockSpec((1, r, 128), lambda i: (i, 0, 0)),
            pl.BlockSpec((DE, DK), lambda i: (0, 0)),
            pl.BlockSpec((1, DK), lambda i: (0, 0)),
        ],
        out_specs=[
            pl.BlockSpec((EB, DK), lambda i: (i, 0)),
            pl.BlockSpec((1, r, 128), lambda i: (i, 0, 0)),
        ],
        out_shape=[
            jax.ShapeDtypeStruct((E, DK), jnp.float32),
            jax.ShapeDtypeStruct((NGRID_E, r, 128), jnp.float32),
        ],
    )(G, KQg, ev, et3, cg3, Wte, bt.reshape(1, DK))
    return out


SW = 80
DH = 64
SPB = 80
NSB = EPT // SPB
NPT = NP // SC_NS


def _softmax_body(att_hbm, colf_hbm, xt_hbm,
                  s1_out, s2_out, mu_out,
                  m_v, tmp_v, acc_v, mu8f,
                  att_b, col_bA, col_bB, xt_buf, w_bufA, w_bufB, msh, ssh,
                  semA, semB):
    cid = lax.axis_index("c")
    sid = lax.axis_index("s")
    wid = sid * SC_NC + cid
    ebase = wid * EPT
    nbase = sid * NPT
    neg = jnp.float32(-1e30)
    zero16 = jnp.zeros((16,), jnp.float32)

    def fill(i, _):
        m_v[pl.ds(i * 16, 16)] = jnp.full((16,), neg)
        return 0
    lax.fori_loop(0, NP // 16, fill, 0)

    def maxbatch(j, _):
        pltpu.sync_copy(att_hbm.at[pl.ds(ebase + j * SPB, SPB)], att_b)
        pltpu.sync_copy(colf_hbm.at[pl.ds(ebase + j * SPB, SPB)], col_bA)

        def rmw(i, _):
            sl = pl.ds(i * 16, 16)
            a16 = att_b[sl]
            c16 = col_bA[sl]
            cur = plsc.load_gather(m_v, [c16])

            def cond(cur):
                return jnp.any(a16 > cur)

            def body(cur):
                plsc.store_scatter(m_v, [c16], a16, mask=a16 > cur)
                return plsc.load_gather(m_v, [c16])

            lax.while_loop(cond, body, cur)
            return 0
        lax.fori_loop(0, SPB // 16, rmw, 0)
        return 0
    lax.fori_loop(0, NSB, maxbatch, 0)

    pltpu.sync_copy(m_v, msh.at[sid])
    plsc.subcore_barrier()
    pltpu.sync_copy(msh.at[0, pl.ds(nbase, NPT)], acc_v)

    def comb(j, _):
        pltpu.sync_copy(msh.at[j, pl.ds(nbase, NPT)], tmp_v)

        def mx(k, _):
            sl = pl.ds(k * 16, 16)
            acc_v[sl] = jnp.maximum(acc_v[sl], tmp_v[sl])
            return 0
        lax.fori_loop(0, NPT // 16, mx, 0)
        return 0
    lax.fori_loop(1, SC_NS, comb, 0)

    def z8(k, _):
        mu8f[pl.ds(k * 16, 16)] = zero16
        return 0
    lax.fori_loop(0, NPT * 8 // 16, z8, 0)

    def sc8(k, _):
        v = acc_v[pl.ds(k * 16, 16)]
        idx = (lax.iota(jnp.int32, 16) + k * 16) * 8
        plsc.store_scatter(mu8f, [idx], v)
        return 0
    lax.fori_loop(0, NPT // 16, sc8, 0)
    pltpu.sync_copy(mu8f, mu_out.at[cid, pl.ds(nbase * 8, NPT * 8)])

    pltpu.sync_copy(acc_v, msh.at[SC_NS, pl.ds(nbase, NPT)])
    plsc.subcore_barrier()
    pltpu.sync_copy(msh.at[SC_NS], m_v)

    for p in range(2):
        out_hbm = s1_out if p == 0 else s2_out

        def zw(k, _):
            e = k // (SW // 16)
            q = k % (SW // 16)
            w_bufA[e, pl.ds(q * 16, 16)] = zero16
            w_bufB[e, pl.ds(q * 16, 16)] = zero16
            return 0
        lax.fori_loop(0, SPB * (SW // 16), zw, 0)

        def zs(k, _):
            pltpu.sync_copy(w_bufA.at[pl.ds(0, 80)],
                            ssh.at[pl.ds(nbase + k * 80, 80), :])
            return 0
        lax.fori_loop(0, NPT // 80, zs, 0)

        plsc.subcore_barrier()

        def build(j, colb, wbuf):
            eoff = ebase + j * SPB
            pltpu.sync_copy(att_hbm.at[pl.ds(eoff, SPB)], att_b)
            pltpu.sync_copy(colf_hbm.at[pl.ds(eoff, SPB)], colb)
            pltpu.sync_copy(
                xt_hbm.at[pl.ds(eoff, SPB), pl.ds(p * DH, DH)], xt_buf)

            def ex(i, _):
                sl = pl.ds(i * 16, 16)
                mu16 = plsc.load_gather(m_v, [colb[sl]])
                att_b[sl] = jnp.exp(att_b[sl] - mu16)
                return 0
            lax.fori_loop(0, SPB // 16, ex, 0)

            if p == 0:
                def edge(e, _):
                    ov = plsc.load_gather(
                        att_b, [jnp.full((16,), e, dtype=jnp.int32)])
                    for d in range(DH // 16):
                        sl = pl.ds(d * 16, 16)
                        wbuf[e, sl] = xt_buf[e, sl] * ov
                    wbuf[e, pl.ds(DH, 16)] = jnp.where(
                        lax.iota(jnp.int32, 16) == 0, ov, 0.0)
                    return 0
            else:
                def edge(e, _):
                    ov = plsc.load_gather(
                        att_b, [jnp.full((16,), e, dtype=jnp.int32)])
                    for d in range(DH // 16):
                        sl = pl.ds(d * 16, 16)
                        wbuf[e, sl] = xt_buf[e, sl] * ov
                    return 0
            lax.fori_loop(0, SPB, edge, 0)

        def batch2(k, _):
            @pl.when(k > 0)
            def _():
                pltpu.make_async_copy(w_bufB, ssh.at[col_bB], semB).wait()
            build(2 * k, col_bA, w_bufA)
            cpA = pltpu.async_copy(w_bufA, ssh.at[col_bA], semA, add=True)
            build(2 * k + 1, col_bB, w_bufB)
            cpA.wait()
            pltpu.async_copy(w_bufB, ssh.at[col_bB], semB, add=True)
            return 0
        lax.fori_loop(0, NSB // 2, batch2, 0)
        pltpu.make_async_copy(w_bufB, ssh.at[col_bB], semB).wait()
        build(NSB - 1, col_bA, w_bufA)
        pltpu.async_copy(w_bufA, ssh.at[col_bA], semA, add=True).wait()

        plsc.subcore_barrier()
        pltpu.sync_copy(ssh.at[pl.ds(nbase, NPT), :],
                        out_hbm.at[cid, pl.ds(nbase, NPT), :])
        plsc.subcore_barrier()


def _sc_softmax_scatter(att, colf, xT):
    mesh = plsc.VectorSubcoreMesh(core_axis_name="c", subcore_axis_name="s",
                                  num_cores=SC_NC, num_subcores=SC_NS)
    f = pl.kernel(
        _softmax_body,
        out_type=[
            jax.ShapeDtypeStruct((SC_NC, NP, SW), jnp.float32),
            jax.ShapeDtypeStruct((SC_NC, NP, SW), jnp.float32),
            jax.ShapeDtypeStruct((SC_NC, NP * 8), jnp.float32),
        ],
        mesh=mesh,
        compiler_params=pltpu.CompilerParams(
            needs_layout_passes=False, use_tc_tiling_on_sc=False),
        scratch_types=[
            pltpu.VMEM((NP,), jnp.float32),
            pltpu.VMEM((NPT,), jnp.float32),
            pltpu.VMEM((NPT,), jnp.float32),
            pltpu.VMEM((NPT * 8,), jnp.float32),
            pltpu.VMEM((SPB,), jnp.float32),
            pltpu.VMEM((SPB,), jnp.int32),
            pltpu.VMEM((SPB,), jnp.int32),
            pltpu.VMEM((SPB, DH), jnp.float32),
            pltpu.VMEM((SPB, SW), jnp.float32),
            pltpu.VMEM((SPB, SW), jnp.float32),
            pltpu.VMEM_SHARED((SC_NS + 1, NP), jnp.float32),
            pltpu.VMEM_SHARED((NP, SW), jnp.float32),
            pltpu.SemaphoreType.DMA,
            pltpu.SemaphoreType.DMA,
        ],
    )
    return f(att, colf, xT)


def _final_kernel(s1_ref, s2_ref, mu_ref, x_ref, wv_ref, bv_ref, out_ref):
    mu0 = mu_ref[0, :, 0:1]
    mu1 = mu_ref[1, :, 0:1]
    mu = jnp.maximum(mu0, mu1)
    r0 = jnp.exp(mu0 - mu)
    r1 = jnp.exp(mu1 - mu)
    s = r0 * s1_ref[0, :, DH:DH + 1] + r1 * s1_ref[1, :, DH:DH + 1]
    inv = 1.0 / (s + 1e-16)
    P1 = (r0 * inv) * s1_ref[0, :, 0:DH] + (r1 * inv) * s1_ref[1, :, 0:DH]
    P2 = (r0 * inv) * s2_ref[0, :, 0:DH] + (r1 * inv) * s2_ref[1, :, 0:DH]
    aggr = jnp.dot(P1, wv_ref[0:DH, :], preferred_element_type=jnp.float32) \
        + jnp.dot(P2, wv_ref[DH:DK, :], preferred_element_type=jnp.float32) \
        + (s * inv) * bv_ref[...]
    out_ref[...] = x_ref[...] + _gelu(aggr)


def _final(S1, S2, mu8, xp, Wv, bv):
    grid = (NGRID_N,)
    return pl.pallas_call(
        _final_kernel,
        grid=grid,
        in_specs=[
            pl.BlockSpec((2, NB, SW), lambda i: (0, i, 0)),
            pl.BlockSpec((2, NB, SW), lambda i: (0, i, 0)),
            pl.BlockSpec((2, NB, 8), lambda i: (0, i, 0)),
            pl.BlockSpec((NB, D), lambda i: (i, 0)),
            pl.BlockSpec((DK, DK), lambda i: (0, 0)),
            pl.BlockSpec((1, DK), lambda i: (0, 0)),
        ],
        out_specs=pl.BlockSpec((NB, D), lambda i: (i, 0)),
        out_shape=jax.ShapeDtypeStruct((NP, D), jnp.float32),
    )(S1, S2, mu8, xp, Wv, bv.reshape(1, DK))


def kernel(x, edge_index, edge_type, edge_vector, ln_g, ln_b, Wt, bt, Wk, bk,
           Wq, bq, Wv, bv):
    row = edge_index[0]
    col = edge_index[1]
    xp = jnp.pad(x, ((0, NP - N), (0, 0)))
    A, kq, c = _node_precompute(xp, ln_g, ln_b, Wt[:D], Wq, bq, Wk.T, bk)

    G, KQg, cg = _sc_gather(A, kq, c, row, col)
    r = EB // 128
    cg3 = cg.reshape(NGRID_E, r, 128)

    et3 = edge_type.reshape(NGRID_E, r, 128)
    xT, att3 = _edge_elementwise(G, KQg, edge_vector, et3, cg3, Wt[D:], bt)
    att = att3.reshape(E)

    S1, S2, mu8f = _sc_softmax_scatter(att, col, xT)

    out = _final(S1, S2, mu8f.reshape(SC_NC, NP, 8), xp, Wv, bv)
    return out[:N]
